# baseline (speedup 1.0000x reference)
import jax
import jax.numpy as jnp
from jax.experimental import pallas as pl
from jax.experimental.pallas import tpu as pltpu

NODE_DIM = 256
FINAL_DIM = 1024
MAX_NEIGHBORS = 49
NUM_EDGE_TYPES = 1000
BF = jnp.bfloat16
F32 = jnp.float32


def _dot(a, b):
    return jnp.dot(a, b, preferred_element_type=F32)


def _front_body(qt_ref, ipw1_ref, ipb1_ref, ipw2_ref, skw_ref, preb_ref,
                lng_ref, lnb_ref, cpw1_ref, cpb1_ref, cpw2_ref, cpb2_ref,
                ngw1_ref, ngb1_ref, nnw1_ref, nnb1_ref, nnw2_ref, nnb2_ref,
                epw1t_ref, epb1_ref,
                ne_ref, cf_ref, nn_ref, ngh_ref, cep_ref):
    qt = qt_ref[...]
    avg = (qt[:, 0, :].astype(F32) + qt[:, 1, :].astype(F32)
           + qt[:, 2, :].astype(F32)) * (1.0 / 3.0)
    avg_b = avg.astype(BF)
    h1 = jnp.maximum(_dot(avg_b, ipw1_ref[...]) + ipb1_ref[...], 0.0)
    proj = _dot(h1.astype(BF), ipw2_ref[...])
    skip = _dot(avg_b, skw_ref[...])
    pre = proj + skip + preb_ref[...]
    mu = jnp.mean(pre, axis=-1, keepdims=True)
    var = jnp.mean(pre * pre, axis=-1, keepdims=True) - mu * mu
    ne = (pre - mu) * jax.lax.rsqrt(var + 1e-5) * lng_ref[...] + lnb_ref[...]
    ne_ref[...] = ne
    ne_b = ne.astype(BF)
    ch = jnp.maximum(_dot(ne_b, cpw1_ref[...]) + cpb1_ref[...], 0.0)
    cf = _dot(ch.astype(BF), cpw2_ref[...]) + cpb2_ref[...]
    cf_ref[...] = cf
    ngh = jnp.maximum(_dot(ne_b, ngw1_ref[...]) + ngb1_ref[...], 0.0)
    ngh_ref[...] = ngh.astype(BF)
    nh = jnp.maximum(_dot(ne_b, nnw1_ref[...]) + nnb1_ref[...], 0.0)
    nn_ref[...] = _dot(nh.astype(BF), nnw2_ref[...]) + nnb2_ref[...]
    cep_ref[...] = _dot(cf.astype(BF), epw1t_ref[...]) + epb1_ref[...]


def _back_body(ngh_ref, cep_ref, w2_ref, b2_ref, epw1b_ref, epw2t_ref,
               epb2t_ref, nf_ref, etl_ref):
    ngh = ngh_ref[...]
    cep = cep_ref[...]
    nf = _dot(ngh, w2_ref[...].astype(BF)) + b2_ref[0]
    nf_ref[0] = nf
    eh = jnp.maximum(_dot(nf.astype(BF), epw1b_ref[...]) + cep, 0.0)
    etl = jax.lax.dot_general(
        epw2t_ref[...], eh.astype(BF), (((1,), (1,)), ((), ())),
        preferred_element_type=F32) + epb2t_ref[...]
    etl_ref[0] = etl


def kernel(quantized_tokens, quantized_indices, ip_w1, ip_b1, ip_w2, ip_b2,
           skip_w, skip_b, ln_g, ln_b, cp_w1, cp_b1, cp_w2, cp_b2,
           ng_w1, ng_b1, ng_w2, ng_b2, ep_w1, ep_b1, ep_w2, ep_b2,
           nn_w1, nn_b1, nn_w2, nn_b2):
    del quantized_indices
    B = quantized_tokens.shape[0]
    D = NODE_DIM
    N = MAX_NEIGHBORS
    E = NUM_EDGE_TYPES

    row = lambda v: v.reshape(1, -1).astype(F32)
    qt2 = quantized_tokens.astype(BF)
    ipw1 = ip_w1.astype(BF)
    ipw2 = ip_w2.astype(BF)
    skw = skip_w.astype(BF)
    cpw1 = cp_w1.astype(BF)
    cpw2 = cp_w2.astype(BF)
    ngw1 = ng_w1.astype(BF)
    epw1t = ep_w1[:D].astype(BF)
    epw1b = ep_w1[D:].astype(BF)
    epw2 = ep_w2.astype(BF)
    nnw1 = nn_w1.astype(BF)
    nnw2 = nn_w2.astype(BF)

    ne, cf, nn_logits, ngh, cep = pl.pallas_call(
        _front_body,
        out_shape=[
            jax.ShapeDtypeStruct((B, D), F32),
            jax.ShapeDtypeStruct((B, D), F32),
            jax.ShapeDtypeStruct((B, N + 1), F32),
            jax.ShapeDtypeStruct((B, 2 * D), BF),
            jax.ShapeDtypeStruct((B, D), F32),
        ],
    )(qt2, ipw1, row(ip_b1), ipw2, skw, row(ip_b2 + skip_b),
      row(ln_g), row(ln_b), cpw1, row(cp_b1), cpw2, row(cp_b2),
      ngw1, row(ng_b1), nnw1, row(nn_b1), nnw2, row(nn_b2),
      epw1t, row(ep_b1))

    nf_t, etl_t = pl.pallas_call(
        _back_body,
        grid=(N,),
        compiler_params=pltpu.CompilerParams(
            dimension_semantics=("parallel",)),
        in_specs=[
            pl.BlockSpec((B, 2 * D), lambda n: (0, 0)),
            pl.BlockSpec((B, D), lambda n: (0, 0)),
            pl.BlockSpec((2 * D, D), lambda n: (0, n)),
            pl.BlockSpec((1, 1, D), lambda n: (n, 0, 0)),
            pl.BlockSpec((D, D), lambda n: (0, 0)),
            pl.BlockSpec((E, D), lambda n: (0, 0)),
            pl.BlockSpec((E, 1), lambda n: (0, 0)),
        ],
        out_specs=[
            pl.BlockSpec((1, B, D), lambda n: (n, 0, 0)),
            pl.BlockSpec((1, E, B), lambda n: (n, 0, 0)),
        ],
        out_shape=[
            jax.ShapeDtypeStruct((N, B, D), F32),
            jax.ShapeDtypeStruct((N, E, B), F32),
        ],
    )(ngh, cep, ng_w2, ng_b2.reshape(N, 1, D).astype(F32), epw1b,
      ep_w2.T.astype(BF), ep_b2.reshape(E, 1).astype(F32))

    neighbor_features = jnp.transpose(nf_t, (1, 0, 2))
    edge_type_logits = jnp.transpose(etl_t, (2, 0, 1))
    return (ne, cf, neighbor_features, edge_type_logits, nn_logits)


# speedup vs baseline: 1.1988x; 1.1988x over previous
import jax
import jax.numpy as jnp
from jax.experimental import pallas as pl
from jax.experimental.pallas import tpu as pltpu

NODE_DIM = 256
FINAL_DIM = 1024
MAX_NEIGHBORS = 49
NUM_EDGE_TYPES = 1000
BF = jnp.bfloat16
F32 = jnp.float32


def _dot(a, b):
    return jnp.dot(a, b, preferred_element_type=F32)


def _front_body(qt_ref, ipw1_ref, ipb1_ref, ipw2_ref, skw_ref, preb_ref,
                lng_ref, lnb_ref, cpw1_ref, cpb1_ref, cpw2_ref, cpb2_ref,
                ngw1_ref, ngb1_ref, nnw1_ref, nnb1_ref, nnw2_ref, nnb2_ref,
                epw1t_ref, epb1_ref,
                ne_ref, cf_ref, nn_ref, ngh_ref, cep_ref):
    qt = qt_ref[...]
    F = FINAL_DIM
    avg = (qt[:, :F].astype(F32) + qt[:, F:2 * F].astype(F32)
           + qt[:, 2 * F:].astype(F32)) * (1.0 / 3.0)
    avg_b = avg.astype(BF)
    bf = lambda r: r[...].astype(BF)
    h1 = jnp.maximum(_dot(avg_b, bf(ipw1_ref)) + ipb1_ref[...], 0.0)
    proj = _dot(h1.astype(BF), bf(ipw2_ref))
    skip = _dot(avg_b, bf(skw_ref))
    pre = proj + skip + preb_ref[...]
    mu = jnp.mean(pre, axis=-1, keepdims=True)
    var = jnp.mean(pre * pre, axis=-1, keepdims=True) - mu * mu
    ne = (pre - mu) * jax.lax.rsqrt(var + 1e-5) * lng_ref[...] + lnb_ref[...]
    ne_ref[...] = ne
    ne_b = ne.astype(BF)
    ch = jnp.maximum(_dot(ne_b, bf(cpw1_ref)) + cpb1_ref[...], 0.0)
    cf = _dot(ch.astype(BF), bf(cpw2_ref)) + cpb2_ref[...]
    cf_ref[...] = cf
    ngh = jnp.maximum(_dot(ne_b, bf(ngw1_ref)) + ngb1_ref[...], 0.0)
    ngh_ref[...] = ngh.astype(BF)
    nh = jnp.maximum(_dot(ne_b, bf(nnw1_ref)) + nnb1_ref[...], 0.0)
    nn_ref[...] = _dot(nh.astype(BF), bf(nnw2_ref)) + nnb2_ref[...]
    cep_ref[...] = _dot(cf.astype(BF),
                        epw1t_ref[:NODE_DIM].astype(BF)) + epb1_ref[...]


def _back_body(ngh_ref, cep_ref, w2_ref, b2_ref, epw1b_ref, epw2_ref,
               epb2t_ref, nf_ref, etl_ref):
    ngh = ngh_ref[...]
    cep = cep_ref[...]
    nf = _dot(ngh, w2_ref[...].astype(BF)) + b2_ref[0]
    nf_ref[0] = nf
    eh = jnp.maximum(
        _dot(nf.astype(BF), epw1b_ref[NODE_DIM:].astype(BF)) + cep, 0.0)
    etl = jax.lax.dot_general(
        epw2_ref[...].astype(BF), eh.astype(BF), (((0,), (1,)), ((), ())),
        preferred_element_type=F32) + epb2t_ref[...]
    etl_ref[0] = etl


def kernel(quantized_tokens, quantized_indices, ip_w1, ip_b1, ip_w2, ip_b2,
           skip_w, skip_b, ln_g, ln_b, cp_w1, cp_b1, cp_w2, cp_b2,
           ng_w1, ng_b1, ng_w2, ng_b2, ep_w1, ep_b1, ep_w2, ep_b2,
           nn_w1, nn_b1, nn_w2, nn_b2):
    del quantized_indices
    B = quantized_tokens.shape[0]
    D = NODE_DIM
    N = MAX_NEIGHBORS
    E = NUM_EDGE_TYPES

    row = lambda v: v.reshape(1, -1).astype(F32)
    qt2 = quantized_tokens.astype(BF).reshape(B, 3 * FINAL_DIM)

    ne, cf, nn_logits, ngh, cep = pl.pallas_call(
        _front_body,
        out_shape=[
            jax.ShapeDtypeStruct((B, D), F32),
            jax.ShapeDtypeStruct((B, D), F32),
            jax.ShapeDtypeStruct((B, N + 1), F32),
            jax.ShapeDtypeStruct((B, 2 * D), BF),
            jax.ShapeDtypeStruct((B, D), F32),
        ],
    )(qt2, ip_w1, row(ip_b1), ip_w2, skip_w, row(ip_b2 + skip_b),
      row(ln_g), row(ln_b), cp_w1, row(cp_b1), cp_w2, row(cp_b2),
      ng_w1, row(ng_b1), nn_w1, row(nn_b1), nn_w2, row(nn_b2),
      ep_w1, row(ep_b1))

    nf_t, etl_t = pl.pallas_call(
        _back_body,
        grid=(N,),
        compiler_params=pltpu.CompilerParams(
            dimension_semantics=("parallel",)),
        in_specs=[
            pl.BlockSpec((B, 2 * D), lambda n: (0, 0)),
            pl.BlockSpec((B, D), lambda n: (0, 0)),
            pl.BlockSpec((2 * D, D), lambda n: (0, n)),
            pl.BlockSpec((1, 1, D), lambda n: (n, 0, 0)),
            pl.BlockSpec((2 * D, D), lambda n: (0, 0)),
            pl.BlockSpec((D, E), lambda n: (0, 0)),
            pl.BlockSpec((E, 1), lambda n: (0, 0)),
        ],
        out_specs=[
            pl.BlockSpec((1, B, D), lambda n: (n, 0, 0)),
            pl.BlockSpec((1, E, B), lambda n: (n, 0, 0)),
        ],
        out_shape=[
            jax.ShapeDtypeStruct((N, B, D), F32),
            jax.ShapeDtypeStruct((N, E, B), F32),
        ],
    )(ngh, cep, ng_w2, ng_b2.reshape(N, 1, D).astype(F32), ep_w1,
      ep_w2, ep_b2.reshape(E, 1).astype(F32))

    neighbor_features = jnp.transpose(nf_t, (1, 0, 2))
    edge_type_logits = jnp.transpose(etl_t, (2, 0, 1))
    return (ne, cf, neighbor_features, edge_type_logits, nn_logits)


# speedup vs baseline: 1.2275x; 1.0239x over previous
import jax
import jax.numpy as jnp
from jax.experimental import pallas as pl
from jax.experimental.pallas import tpu as pltpu

NODE_DIM = 256
FINAL_DIM = 1024
MAX_NEIGHBORS = 49
NUM_EDGE_TYPES = 1000
BF = jnp.bfloat16
F32 = jnp.float32


def _dot(a, b):
    return jnp.dot(a, b, preferred_element_type=F32)


def _full_body(qt_ref, ipw1_ref, ipb1_ref, ipw2_ref, skw_ref, preb_ref,
               lng_ref, lnb_ref, cpw1_ref, cpb1_ref, cpw2_ref, cpb2_ref,
               ngw1_ref, ngb1_ref, nnw1_ref, nnb1_ref, nnw2_ref, nnb2_ref,
               epw1_ref, epb1_ref, w2_ref, b2_ref, epw2_ref, epb2t_ref,
               ne_ref, cf_ref, nn_ref, nf_ref, etl_ref, ngh_s, cep_s):
    n = pl.program_id(0)
    D = NODE_DIM
    bf = lambda r: r[...].astype(BF)

    @pl.when(n == 0)
    def _front():
        qt = qt_ref[...]
        F = FINAL_DIM
        avg = (qt[:, :F].astype(F32) + qt[:, F:2 * F].astype(F32)
               + qt[:, 2 * F:].astype(F32)) * (1.0 / 3.0)
        avg_b = avg.astype(BF)
        h1 = jnp.maximum(_dot(avg_b, bf(ipw1_ref)) + ipb1_ref[...], 0.0)
        proj = _dot(h1.astype(BF), bf(ipw2_ref))
        skip = _dot(avg_b, bf(skw_ref))
        pre = proj + skip + preb_ref[...]
        mu = jnp.mean(pre, axis=-1, keepdims=True)
        var = jnp.mean(pre * pre, axis=-1, keepdims=True) - mu * mu
        ne = (pre - mu) * jax.lax.rsqrt(var + 1e-5) * lng_ref[...] \
            + lnb_ref[...]
        ne_ref[...] = ne
        ne_b = ne.astype(BF)
        ch = jnp.maximum(_dot(ne_b, bf(cpw1_ref)) + cpb1_ref[...], 0.0)
        cf = _dot(ch.astype(BF), bf(cpw2_ref)) + cpb2_ref[...]
        cf_ref[...] = cf
        ngh_s[...] = jnp.maximum(
            _dot(ne_b, bf(ngw1_ref)) + ngb1_ref[...], 0.0).astype(BF)
        nh = jnp.maximum(_dot(ne_b, bf(nnw1_ref)) + nnb1_ref[...], 0.0)
        nn_ref[...] = _dot(nh.astype(BF), bf(nnw2_ref)) + nnb2_ref[...]
        cep_s[...] = _dot(cf.astype(BF),
                          epw1_ref[:D].astype(BF)) + epb1_ref[...]

    ngh = ngh_s[...]
    cep = cep_s[...]
    nf = _dot(ngh, w2_ref[...].astype(BF)) + b2_ref[0]
    nf_ref[0] = nf
    eh = jnp.maximum(
        _dot(nf.astype(BF), epw1_ref[D:].astype(BF)) + cep, 0.0)
    etl = jax.lax.dot_general(
        epw2_ref[...].astype(BF), eh.astype(BF), (((0,), (1,)), ((), ())),
        preferred_element_type=F32) + epb2t_ref[...]
    etl_ref[0] = etl


def kernel(quantized_tokens, quantized_indices, ip_w1, ip_b1, ip_w2, ip_b2,
           skip_w, skip_b, ln_g, ln_b, cp_w1, cp_b1, cp_w2, cp_b2,
           ng_w1, ng_b1, ng_w2, ng_b2, ep_w1, ep_b1, ep_w2, ep_b2,
           nn_w1, nn_b1, nn_w2, nn_b2):
    del quantized_indices
    B = quantized_tokens.shape[0]
    D = NODE_DIM
    N = MAX_NEIGHBORS
    E = NUM_EDGE_TYPES

    row = lambda v: v.reshape(1, -1).astype(F32)
    qt2 = quantized_tokens.astype(BF).reshape(B, 3 * FINAL_DIM)

    F = FINAL_DIM
    full = lambda shape: pl.BlockSpec(shape, lambda n: (0,) * len(shape))
    ne, cf, nn_logits, nf_t, etl_t = pl.pallas_call(
        _full_body,
        grid=(N,),
        in_specs=[
            full((B, 3 * F)), full((F, 4 * D)), full((1, 4 * D)),
            full((4 * D, D)), full((F, D)), full((1, D)),
            full((1, D)), full((1, D)), full((D, D)), full((1, D)),
            full((D, D)), full((1, D)), full((D, 2 * D)), full((1, 2 * D)),
            full((D, D // 2)), full((1, D // 2)), full((D // 2, N + 1)),
            full((1, N + 1)), full((2 * D, D)), full((1, D)),
            pl.BlockSpec((2 * D, D), lambda n: (0, n)),
            pl.BlockSpec((1, 1, D), lambda n: (n, 0, 0)),
            full((D, E)), full((E, 1)),
        ],
        out_specs=[
            full((B, D)), full((B, D)), full((B, N + 1)),
            pl.BlockSpec((1, B, D), lambda n: (n, 0, 0)),
            pl.BlockSpec((1, E, B), lambda n: (n, 0, 0)),
        ],
        out_shape=[
            jax.ShapeDtypeStruct((B, D), F32),
            jax.ShapeDtypeStruct((B, D), F32),
            jax.ShapeDtypeStruct((B, N + 1), F32),
            jax.ShapeDtypeStruct((N, B, D), F32),
            jax.ShapeDtypeStruct((N, E, B), F32),
        ],
        scratch_shapes=[
            pltpu.VMEM((B, 2 * D), BF),
            pltpu.VMEM((B, D), F32),
        ],
    )(qt2, ip_w1, row(ip_b1), ip_w2, skip_w, row(ip_b2 + skip_b),
      row(ln_g), row(ln_b), cp_w1, row(cp_b1), cp_w2, row(cp_b2),
      ng_w1, row(ng_b1), nn_w1, row(nn_b1), nn_w2, row(nn_b2),
      ep_w1, row(ep_b1), ng_w2, ng_b2.reshape(N, 1, D).astype(F32),
      ep_w2, ep_b2.reshape(E, 1).astype(F32))

    neighbor_features = jnp.transpose(nf_t, (1, 0, 2))
    edge_type_logits = jnp.transpose(etl_t, (2, 0, 1))
    return (ne, cf, neighbor_features, edge_type_logits, nn_logits)
